# Initial kernel scaffold; baseline (speedup 1.0000x reference)
import jax
import jax.numpy as jnp
from jax.experimental import pallas as pl


def kernel(x, batch, W0, b0, gamma0, beta0, rm0, rv0, W1, b1, gamma1, beta1, rm1, rv1):
    raise NotImplementedError("write your pallas kernel here")


# speedup vs baseline: 6.4541x; 6.4541x over previous
import functools

import jax
import jax.numpy as jnp
from jax import lax
from jax.experimental import pallas as pl
from jax.experimental.pallas import tpu as pltpu
from jax.experimental.pallas import tpu_sc as plsc

_EPS = 1e-5
_K = 16
_G = 8
_R = 128
_C = 512
_RP = 512
_NEG = -1e30
_NW = 32
_CN = 8


def _knn_body(npad, lob_ref, hib_ref, bcol_ref, sqcol_ref, brow_ref, sqrow_ref,
              xfull_ref, xrow_ref, wut_ref, wvt_ref, b0_ref,
              idx_ref, u_ref, v_ref):
    pid = pl.program_id(0)
    xr = xrow_ref[...]
    u_ref[...] = lax.dot_general(xr, wut_ref[...], (((1,), (0,)), ((), ())),
                                 preferred_element_type=jnp.float32) + b0_ref[...]
    vv = lax.dot_general(xr, wvt_ref[...], (((1,), (0,)), ((), ())),
                         preferred_element_type=jnp.float32)
    v_ref[...] = jnp.concatenate(
        [vv, jnp.zeros((vv.shape[0], v_ref.shape[1] - vv.shape[1]), jnp.float32)],
        axis=1)

    lo = lob_ref[pid]
    hi = hib_ref[pid]
    ct0 = lo // _C
    ct1 = (hi + _C - 1) // _C
    extra = jnp.where(ct0 > 0, 1, 0)
    nloops = (ct1 - ct0) + extra

    brow = brow_ref[...]
    sqrow = sqrow_ref[...]
    minf = jnp.float32(-jnp.inf)
    big_i = jnp.int32(npad)

    def tile_step(jj, carry):
        val16, idx16 = carry
        j = jnp.where(jj == 0, 0, ct0 + jj - extra)
        colbase = j * _C
        xc = xfull_ref[pl.ds(colbase, _C), :]
        dot = lax.dot_general(xr, xc, (((1,), (1,)), ((), ())),
                              preferred_element_type=jnp.float32)
        sqc = sqcol_ref[0:1, pl.ds(colbase, _C)]
        bc = bcol_ref[0:1, pl.ds(colbase, _C)]
        negd2 = 2.0 * dot - sqrow - sqc
        cur = jnp.where(brow == bc, negd2, _NEG)
        ii = lax.broadcasted_iota(jnp.int32, (_R, _C), 1) + colbase
        cur_all = jnp.concatenate([cur, val16], axis=1)
        idx_all = jnp.concatenate([ii, idx16], axis=1)
        nv, ni = [], []
        for _ in range(_K):
            m = jnp.max(cur_all, axis=1, keepdims=True)
            pick = cur_all == m
            am = jnp.min(jnp.where(pick, idx_all, big_i), axis=1, keepdims=True)
            nv.append(m)
            ni.append(am)
            cur_all = jnp.where(pick & (idx_all == am), minf, cur_all)
        return jnp.concatenate(nv, axis=1), jnp.concatenate(ni, axis=1)

    val0 = jnp.full((_R, _K), minf, jnp.float32)
    idx0 = jnp.full((_R, _K), big_i, jnp.int32)
    _, idx16 = lax.fori_loop(0, nloops, tile_step, (val0, idx0))
    idx_ref[...] = idx16


def _sc_body(npad, gdim, v_hbm, idxf_hbm, mx_hbm, mn_hbm,
             idx_v, rows_v, omx_v, omn_v, sem):
    c = lax.axis_index("c")
    s = lax.axis_index("s")
    wid = s * 2 + c
    rows_per_w = npad // _NW
    nchunks = rows_per_w // _CN
    nvec = gdim // 16

    def chunk(ci, carry):
        nodebase = wid * rows_per_w + ci * _CN
        pltpu.sync_copy(idxf_hbm.at[pl.ds(nodebase * _K, _CN * _K)], idx_v)
        pltpu.async_copy(v_hbm.at[idx_v], rows_v, sem).wait()

        def node(n, carry2):
            for cv in range(nvec):
                sl = pl.ds(cv * 16, 16)
                mx = rows_v[n * _K, sl]
                mn = mx
                for k in range(1, _K):
                    w = rows_v[n * _K + k, sl]
                    mx = jnp.maximum(mx, w)
                    mn = jnp.minimum(mn, w)
                omx_v[n, sl] = mx
                omn_v[n, sl] = mn
            return carry2

        lax.fori_loop(0, _CN, node, 0)
        pltpu.sync_copy(omx_v, mx_hbm.at[pl.ds(nodebase, _CN)])
        pltpu.sync_copy(omn_v, mn_hbm.at[pl.ds(nodebase, _CN)])
        return carry

    lax.fori_loop(0, nchunks, chunk, 0)


def _pool_body(u_ref, mx_ref, mn_ref, x_ref, brow_ref, sc_ref, bs_ref,
               hp_ref, xp_ref):
    i = pl.program_id(0)
    minf = jnp.float32(-jnp.inf)
    sc = sc_ref[...]
    h = jnp.maximum(u_ref[...] + jnp.where(sc >= 0.0, mx_ref[...], mn_ref[...]),
                    0.0) * sc + bs_ref[...]
    xb = x_ref[...]
    b = brow_ref[...]

    @pl.when(i == 0)
    def _():
        hp_ref[...] = jnp.full(hp_ref.shape, minf, jnp.float32)
        xp_ref[...] = jnp.full(xp_ref.shape, minf, jnp.float32)

    for g in range(_G):
        m = b == g
        hg = jnp.max(jnp.where(m, h, minf), axis=0, keepdims=True)
        xg = jnp.max(jnp.where(m, xb, minf), axis=0, keepdims=True)
        hp_ref[g:g + 1, :] = jnp.maximum(hp_ref[g:g + 1, :], hg)
        xp_ref[g:g + 1, :] = jnp.maximum(xp_ref[g:g + 1, :], xg)


def kernel(x, batch, W0, b0, gamma0, beta0, rm0, rv0,
           W1, b1, gamma1, beta1, rm1, rv1):
    n, d = x.shape
    g = W0.shape[0]
    npad = -(-n // 2560) * 2560
    pad = npad - n

    xpd = jnp.pad(x, ((0, pad), (0, 0)))
    b32 = batch.astype(jnp.int32)
    bp = jnp.concatenate([b32, jnp.full((pad,), _G, jnp.int32)])
    sq = jnp.sum(xpd * xpd, axis=1)

    nblocks = npad // _R
    rb = jnp.arange(nblocks) * _R
    lob = jnp.searchsorted(bp, bp[rb], side="left").astype(jnp.int32)
    hib = jnp.searchsorted(bp, bp[rb + _R - 1], side="right").astype(jnp.int32)

    wut = (W0[:, :d] - W0[:, d:]).T
    wvt = W0[:, d:].T
    b0r = b0.reshape(1, g)
    scale = (gamma0 / jnp.sqrt(rv0 + _EPS)).reshape(1, g)
    bias = beta0.reshape(1, g) - rm0.reshape(1, g) * scale

    full = lambda *shape: pl.BlockSpec(shape, lambda i: (0,) * len(shape))
    rowblk = lambda blk, w: pl.BlockSpec((blk, w), lambda i: (i, 0))

    idx, u, v = pl.pallas_call(
        functools.partial(_knn_body, npad),
        grid=(nblocks,),
        in_specs=[
            pl.BlockSpec(memory_space=pltpu.SMEM),
            pl.BlockSpec(memory_space=pltpu.SMEM),
            full(1, npad),
            full(1, npad),
            rowblk(_R, 1),
            rowblk(_R, 1),
            full(npad, d),
            rowblk(_R, d),
            full(d, g),
            full(d, g),
            full(1, g),
        ],
        out_specs=[rowblk(_R, _K), rowblk(_R, g), rowblk(_R, 128)],
        out_shape=[
            jax.ShapeDtypeStruct((npad, _K), jnp.int32),
            jax.ShapeDtypeStruct((npad, g), jnp.float32),
            jax.ShapeDtypeStruct((npad, 128), jnp.float32),
        ],
    )(lob, hib, bp.reshape(1, npad), sq.reshape(1, npad),
      bp.reshape(npad, 1), sq.reshape(npad, 1), xpd, xpd, wut, wvt, b0r)

    mesh = plsc.VectorSubcoreMesh(core_axis_name="c", subcore_axis_name="s")
    mx, mn = pl.kernel(
        functools.partial(_sc_body, npad, g),
        out_type=(jax.ShapeDtypeStruct((npad, g), jnp.float32),
                  jax.ShapeDtypeStruct((npad, g), jnp.float32)),
        mesh=mesh,
        scratch_types=[
            pltpu.VMEM((_CN * _K,), jnp.int32),
            pltpu.VMEM((_CN * _K, 128), jnp.float32),
            pltpu.VMEM((_CN, g), jnp.float32),
            pltpu.VMEM((_CN, g), jnp.float32),
            pltpu.SemaphoreType.DMA,
        ],
    )(v, idx.reshape(npad * _K))

    hp, xp = pl.pallas_call(
        _pool_body,
        grid=(npad // _RP,),
        in_specs=[
            rowblk(_RP, g),
            rowblk(_RP, g),
            rowblk(_RP, g),
            rowblk(_RP, d),
            rowblk(_RP, 1),
            full(1, g),
            full(1, g),
        ],
        out_specs=[full(_G, g), full(_G, d)],
        out_shape=[
            jax.ShapeDtypeStruct((_G, g), jnp.float32),
            jax.ShapeDtypeStruct((_G, d), jnp.float32),
        ],
    )(u, mx, mn, xpd, bp.reshape(npad, 1), scale, bias)

    return jnp.concatenate([hp, xp], axis=1)
